# baseline (speedup 1.0000x reference)
import jax
import jax.numpy as jnp
from jax.experimental import pallas as pl
from jax.experimental.pallas import tpu as pltpu


def _copy_body(hp_ref, qt_ref, hp_out, qt_out, hp_sem, qt_sem):
    hp_dma = pltpu.make_async_copy(hp_ref, hp_out, hp_sem)
    qt_dma = pltpu.make_async_copy(qt_ref, qt_out, qt_sem)
    hp_dma.start()
    qt_dma.start()
    hp_dma.wait()
    qt_dma.wait()


def kernel(hip_pos, quat):
    hp_o, qt_o = pl.pallas_call(
        _copy_body,
        in_specs=[
            pl.BlockSpec(memory_space=pl.ANY),
            pl.BlockSpec(memory_space=pl.ANY),
        ],
        out_specs=[
            pl.BlockSpec(memory_space=pl.ANY),
            pl.BlockSpec(memory_space=pl.ANY),
        ],
        out_shape=[
            jax.ShapeDtypeStruct(hip_pos.shape, hip_pos.dtype),
            jax.ShapeDtypeStruct(quat.shape, quat.dtype),
        ],
        scratch_shapes=[pltpu.SemaphoreType.DMA, pltpu.SemaphoreType.DMA],
    )(hip_pos, quat)
    return hp_o, qt_o


# speedup vs baseline: 28.2769x; 28.2769x over previous
import jax
import jax.numpy as jnp
from jax.experimental import pallas as pl
from jax.experimental.pallas import tpu as pltpu


def _copy_body(hp_ref, qt_ref, hp_out, qt_out, hp_sem, qt_sem):
    hp_dma = pltpu.make_async_copy(hp_ref, hp_out, hp_sem)
    qt_dma = pltpu.make_async_copy(qt_ref, qt_out, qt_sem)
    hp_dma.start()
    qt_dma.start()
    hp_dma.wait()
    qt_dma.wait()


def kernel(hip_pos, quat):
    hp = hip_pos.reshape(1024, 128 * 1 * 3)
    qt = quat.reshape(1024, 128 * 52 * 4)
    hp_o, qt_o = pl.pallas_call(
        _copy_body,
        in_specs=[
            pl.BlockSpec(memory_space=pl.ANY),
            pl.BlockSpec(memory_space=pl.ANY),
        ],
        out_specs=[
            pl.BlockSpec(memory_space=pl.ANY),
            pl.BlockSpec(memory_space=pl.ANY),
        ],
        out_shape=[
            jax.ShapeDtypeStruct(hp.shape, hp.dtype),
            jax.ShapeDtypeStruct(qt.shape, qt.dtype),
        ],
        scratch_shapes=[pltpu.SemaphoreType.DMA, pltpu.SemaphoreType.DMA],
    )(hp, qt)
    return hp_o.reshape(hip_pos.shape), qt_o.reshape(quat.shape)
